# baseline (speedup 1.0000x reference)
import functools

import jax
import jax.numpy as jnp
from jax import lax
from jax.experimental import pallas as pl
from jax.experimental.pallas import tpu as pltpu
from jax.experimental.pallas import tpu_sc as plsc

B = 8
N = 8192
D = 768
K = 100
C = 20
ITERS = 25
ROWS = 64
LANES = 128
KP = 112

_SC_MESH = plsc.VectorSubcoreMesh(core_axis_name="c", subcore_axis_name="s")


def _topk_body(a0_ref, a1_ref, idx_ref, score_ref):
    x0 = a0_ref[...]
    x1 = a1_ref[...]
    m = jnp.maximum(x0, x1)
    e0 = jnp.exp(x0 - m)
    e1 = jnp.exp(x1 - m)
    score_ref[...] = e1 / (e0 + e1)
    lin = (lax.broadcasted_iota(jnp.int32, (B, ROWS, LANES), 1) * LANES
           + lax.broadcasted_iota(jnp.int32, (B, ROWS, LANES), 2))
    lane = lax.broadcasted_iota(jnp.int32, (B, LANES), 1)
    boff = lax.broadcasted_iota(jnp.int32, (B, LANES), 0) * N

    def step(j, acc):
        s = score_ref[...]
        mx = jnp.max(jnp.max(s, axis=2, keepdims=True), axis=1,
                     keepdims=True)
        cand = jnp.where(s == mx, lin, jnp.int32(1 << 30))
        idx = jnp.min(jnp.min(cand, axis=2, keepdims=True), axis=1,
                      keepdims=True)
        score_ref[...] = jnp.where(lin == idx, jnp.float32(-1.0), s)
        return jnp.where(lane == j, idx.reshape(B, 1), acc)

    acc = lax.fori_loop(0, K, step, jnp.zeros((B, LANES), jnp.int32))
    idx_ref[...] = (acc + boff)[:, :KP]


@functools.partial(
    pl.kernel,
    out_type=jax.ShapeDtypeStruct((B, KP, D), jnp.float32),
    mesh=_SC_MESH,
    scratch_types=[
        pltpu.VMEM((KP,), jnp.int32),
        pltpu.VMEM((KP, D), jnp.float32),
        pltpu.SemaphoreType.DMA,
    ],
)
def _sc_gather(idx_hbm, ptf_hbm, out_hbm, idx_v, rows_v, sem):
    wid = lax.axis_index("s") * 2 + lax.axis_index("c")

    @pl.when(wid < B)
    def _work():
        pltpu.sync_copy(idx_hbm.at[wid], idx_v)
        pltpu.async_copy(ptf_hbm.at[idx_v], rows_v, sem).wait()
        pltpu.sync_copy(rows_v, out_hbm.at[wid])


def _tc_kmeans_body(sel_ref, out_ref):
    sel = sel_ref[...]
    ones = jnp.ones((B * K, 1), jnp.float32)
    selx = jnp.concatenate([sel, ones], axis=1)
    col = lax.broadcasted_iota(jnp.int32, (B * K, B * C), 1)
    c0 = jnp.concatenate(
        [sel[b * K:b * K + C] for b in range(B)], axis=0)

    G = 4
    BPG = B // G
    colg = lax.broadcasted_iota(jnp.int32, (BPG * K, BPG * C), 1)
    rowbg = lax.broadcasted_iota(jnp.int32, (BPG * K, BPG * C), 0) // K
    validg = (colg // C) == rowbg

    def labels_of(centers):
        c2 = jnp.sum(centers * centers, axis=1, keepdims=True)
        cext = jnp.concatenate([-2.0 * centers, c2], axis=1)
        labs = []
        for g in range(G):
            d = lax.dot_general(
                selx[g * BPG * K:(g + 1) * BPG * K],
                cext[g * BPG * C:(g + 1) * BPG * C],
                (((1,), (1,)), ((), ())),
                precision=lax.Precision.HIGHEST,
                preferred_element_type=jnp.float32)
            d = jnp.where(validg, d, jnp.float32(jnp.inf))
            minv = jnp.min(d, axis=1, keepdims=True)
            labs.append(jnp.min(
                jnp.where(d == minv, colg, jnp.int32(1 << 30)),
                axis=1, keepdims=True) + g * BPG * C)
        return jnp.concatenate(labs, axis=0)

    def moments(lab):
        one = (lab == col).astype(jnp.float32)
        se = lax.dot_general(
            one, selx, (((0,), (0,)), ((), ())),
            preferred_element_type=jnp.float32)
        return se[:, :D], se[:, D:D + 1]

    def cond(st):
        i, _, _, changed = st
        return (i < ITERS) & changed

    def body(st):
        i, centers, lab_prev, _ = st
        lab = labels_of(centers)
        sums, counts = moments(lab)
        newc = jnp.where(counts > 0,
                         sums / jnp.maximum(counts, 1.0), centers)
        return (i + 1, newc, lab, jnp.any(lab != lab_prev))

    st0 = (jnp.int32(0), c0,
           jnp.full((B * K, 1), -1, jnp.int32), jnp.bool_(True))
    _, centers, _, _ = lax.while_loop(cond, body, st0)
    sums, counts = moments(labels_of(centers))
    cf = sums / jnp.maximum(counts, 1.0)
    norm = jnp.sqrt(jnp.sum(cf * cf, axis=1, keepdims=True))
    out_ref[...] = cf / jnp.maximum(norm, jnp.float32(1e-12))


def kernel(patch_token, anomaly_map, prompt_id):
    del prompt_id
    a0 = anomaly_map[:, :, 0].reshape(B, ROWS, LANES)
    a1 = anomaly_map[:, :, 1].reshape(B, ROWS, LANES)
    idx = pl.pallas_call(
        _topk_body,
        out_shape=jax.ShapeDtypeStruct((B, KP), jnp.int32),
        scratch_shapes=[pltpu.VMEM((B, ROWS, LANES), jnp.float32)],
    )(a0, a1)
    ptf = patch_token.reshape(B * N, D)
    sel = _sc_gather(idx, ptf)[:, :K, :]
    out = pl.pallas_call(
        _tc_kmeans_body,
        out_shape=jax.ShapeDtypeStruct((B * C, D), jnp.float32),
    )(sel.reshape(B * K, D))
    return out.reshape(B, C, D)


# speedup vs baseline: 1.3808x; 1.3808x over previous
import jax
import jax.numpy as jnp
from jax import lax
from jax.experimental import pallas as pl
from jax.experimental.pallas import tpu as pltpu

B = 8
N = 8192
D = 768
K = 100
C = 20
ITERS = 25
ROWS = 64
LANES = 128


def _body(a0_ref, a1_ref, pt_ref, out_ref, sel_ref, score_ref, idxv_ref,
          idxs_ref, sem):
    pid = pl.program_id(0)

    @pl.when(pid == 0)
    def _topk_gather():
        x0 = a0_ref[...]
        x1 = a1_ref[...]
        m = jnp.maximum(x0, x1)
        e0 = jnp.exp(x0 - m)
        e1 = jnp.exp(x1 - m)
        score_ref[...] = e1 / (e0 + e1)
        lin = (lax.broadcasted_iota(jnp.int32, (B, ROWS, LANES), 1) * LANES
               + lax.broadcasted_iota(jnp.int32, (B, ROWS, LANES), 2))
        lane = lax.broadcasted_iota(jnp.int32, (B, LANES), 1)

        def step(j, acc):
            s = score_ref[...]
            mx = jnp.max(s, axis=(1, 2), keepdims=True)
            cand = jnp.where(s == mx, lin, jnp.int32(1 << 30))
            idx = jnp.min(cand, axis=(1, 2), keepdims=True)
            score_ref[...] = jnp.where(lin == idx, jnp.float32(-1.0), s)
            return jnp.where(lane == j, idx.reshape(B, 1), acc)

        acc = lax.fori_loop(0, K, step,
                            jnp.zeros((B, LANES), jnp.int32))
        idxv_ref[...] = acc
        pltpu.make_async_copy(idxv_ref, idxs_ref, sem).start()
        pltpu.make_async_copy(idxv_ref, idxs_ref, sem).wait()

        for b in range(B):
            def issue(j, _, b=b):
                idx = idxs_ref[b, j]
                pltpu.make_async_copy(
                    pt_ref.at[b, pl.ds(idx, 1), :],
                    sel_ref.at[pl.ds(b * K + j, 1), :],
                    sem,
                ).start()
                return 0

            lax.fori_loop(0, K, issue, 0)

        def drain(j, _):
            pltpu.make_async_copy(
                pt_ref.at[0, pl.ds(0, 1), :],
                sel_ref.at[pl.ds(0, 1), :],
                sem,
            ).wait()
            return 0

        lax.fori_loop(0, B * K, drain, 0)

    @pl.when(pid == 1)
    def _kmeans():
        sel = sel_ref[...]
        ones = jnp.ones((B * K, 1), jnp.float32)
        selx = jnp.concatenate([sel, ones], axis=1)
        col = lax.broadcasted_iota(jnp.int32, (B * K, B * C), 1)
        rowb = lax.broadcasted_iota(jnp.int32, (B * K, B * C), 0) // K
        valid = (col // C) == rowb
        c0 = jnp.concatenate(
            [sel[b * K:b * K + C] for b in range(B)], axis=0)

        G = 4
        BPG = B // G
        colg = lax.broadcasted_iota(jnp.int32, (BPG * K, BPG * C), 1)
        rowbg = lax.broadcasted_iota(jnp.int32, (BPG * K, BPG * C), 0) // K
        validg = (colg // C) == rowbg

        def labels_of(centers):
            c2 = jnp.sum(centers * centers, axis=1, keepdims=True)
            cext = jnp.concatenate([-2.0 * centers, c2], axis=1)
            labs = []
            for g in range(G):
                d = lax.dot_general(
                    selx[g * BPG * K:(g + 1) * BPG * K],
                    cext[g * BPG * C:(g + 1) * BPG * C],
                    (((1,), (1,)), ((), ())),
                    precision=lax.Precision.HIGHEST,
                    preferred_element_type=jnp.float32)
                d = jnp.where(validg, d, jnp.float32(jnp.inf))
                minv = jnp.min(d, axis=1, keepdims=True)
                labs.append(jnp.min(
                    jnp.where(d == minv, colg, jnp.int32(1 << 30)),
                    axis=1, keepdims=True) + g * BPG * C)
            return jnp.concatenate(labs, axis=0)

        def moments(lab):
            one = (lab == col).astype(jnp.float32)
            se = lax.dot_general(
                one, selx, (((0,), (0,)), ((), ())),
                preferred_element_type=jnp.float32)
            return se[:, :D], se[:, D:D + 1]

        def cond(st):
            i, _, _, changed = st
            return (i < ITERS) & changed

        def body(st):
            i, centers, lab_prev, _ = st
            lab = labels_of(centers)
            sums, counts = moments(lab)
            newc = jnp.where(counts > 0,
                             sums / jnp.maximum(counts, 1.0), centers)
            return (i + 1, newc, lab,
                    jnp.any(lab != lab_prev))

        st0 = (jnp.int32(0), c0,
               jnp.full((B * K, 1), -1, jnp.int32), jnp.bool_(True))
        _, centers, _, _ = lax.while_loop(cond, body, st0)
        sums, counts = moments(labels_of(centers))
        cf = sums / jnp.maximum(counts, 1.0)
        norm = jnp.sqrt(jnp.sum(cf * cf, axis=1, keepdims=True))
        out_ref[...] = cf / jnp.maximum(norm, jnp.float32(1e-12))


def kernel(patch_token, anomaly_map, prompt_id):
    del prompt_id
    a0 = anomaly_map[:, :, 0].reshape(B, ROWS, LANES)
    a1 = anomaly_map[:, :, 1].reshape(B, ROWS, LANES)
    out = pl.pallas_call(
        _body,
        grid=(2,),
        in_specs=[
            pl.BlockSpec((B, ROWS, LANES), lambda i: (0, 0, 0)),
            pl.BlockSpec((B, ROWS, LANES), lambda i: (0, 0, 0)),
            pl.BlockSpec(memory_space=pl.ANY),
        ],
        out_specs=pl.BlockSpec((B * C, D), lambda i: (0, 0)),
        out_shape=jax.ShapeDtypeStruct((B * C, D), jnp.float32),
        scratch_shapes=[
            pltpu.VMEM((B * K, D), jnp.float32),
            pltpu.VMEM((B, ROWS, LANES), jnp.float32),
            pltpu.VMEM((B, LANES), jnp.int32),
            pltpu.SMEM((B, LANES), jnp.int32),
            pltpu.SemaphoreType.DMA,
        ],
    )(a0, a1, patch_token)
    return out.reshape(B, C, D)


# speedup vs baseline: 1.4445x; 1.0462x over previous
import jax
import jax.numpy as jnp
from jax import lax
from jax.experimental import pallas as pl
from jax.experimental.pallas import tpu as pltpu

B = 8
N = 8192
D = 768
K = 100
C = 20
ITERS = 25
ROWS = 64
LANES = 128


def _body(a0_ref, a1_ref, pt_ref, out_ref, sel_ref, score_ref, idxv_ref,
          idxs_ref, sem):
    pid = pl.program_id(0)

    @pl.when(pid == 0)
    def _topk_gather():
        x0 = a0_ref[...]
        x1 = a1_ref[...]
        m = jnp.maximum(x0, x1)
        e0 = jnp.exp(x0 - m)
        e1 = jnp.exp(x1 - m)
        score_ref[...] = e1 / (e0 + e1)
        lin = (lax.broadcasted_iota(jnp.int32, (B, ROWS, LANES), 1) * LANES
               + lax.broadcasted_iota(jnp.int32, (B, ROWS, LANES), 2))
        lane = lax.broadcasted_iota(jnp.int32, (B, LANES), 1)

        def step(j, acc):
            s = score_ref[...]
            mx = jnp.max(s, axis=(1, 2), keepdims=True)
            cand = jnp.where(s == mx, lin, jnp.int32(1 << 30))
            idx = jnp.min(cand, axis=(1, 2), keepdims=True)
            score_ref[...] = jnp.where(lin == idx, jnp.float32(-1.0), s)
            return jnp.where(lane == j, idx.reshape(B, 1), acc)

        acc = lax.fori_loop(0, K, step,
                            jnp.zeros((B, LANES), jnp.int32), unroll=4)
        idxv_ref[...] = acc
        pltpu.make_async_copy(idxv_ref, idxs_ref, sem).start()
        pltpu.make_async_copy(idxv_ref, idxs_ref, sem).wait()

        for b in range(B):
            def issue(j, _, b=b):
                idx = idxs_ref[b, j]
                pltpu.make_async_copy(
                    pt_ref.at[b, pl.ds(idx, 1), :],
                    sel_ref.at[pl.ds(b * K + j, 1), :],
                    sem,
                ).start()
                return 0

            lax.fori_loop(0, K, issue, 0)

        def drain(j, _):
            pltpu.make_async_copy(
                pt_ref.at[0, pl.ds(0, 1), :],
                sel_ref.at[pl.ds(0, 1), :],
                sem,
            ).wait()
            return 0

        lax.fori_loop(0, B * K, drain, 0)

    @pl.when(pid == 1)
    def _kmeans():
        sel = sel_ref[...]
        ones = jnp.ones((B * K, 1), jnp.float32)
        selx = jnp.concatenate([sel, ones], axis=1)
        col = lax.broadcasted_iota(jnp.int32, (B * K, B * C), 1)
        rowb = lax.broadcasted_iota(jnp.int32, (B * K, B * C), 0) // K
        valid = (col // C) == rowb
        c0 = jnp.concatenate(
            [sel[b * K:b * K + C] for b in range(B)], axis=0)

        G = 4
        BPG = B // G
        colg = lax.broadcasted_iota(jnp.int32, (BPG * K, BPG * C), 1)
        rowbg = lax.broadcasted_iota(jnp.int32, (BPG * K, BPG * C), 0) // K
        validg = (colg // C) == rowbg

        def labels_of(centers):
            c2 = jnp.sum(centers * centers, axis=1, keepdims=True)
            cext = jnp.concatenate([-2.0 * centers, c2], axis=1)
            labs = []
            for g in range(G):
                d = lax.dot_general(
                    selx[g * BPG * K:(g + 1) * BPG * K],
                    cext[g * BPG * C:(g + 1) * BPG * C],
                    (((1,), (1,)), ((), ())),
                    precision=lax.Precision.HIGHEST,
                    preferred_element_type=jnp.float32)
                d = jnp.where(validg, d, jnp.float32(jnp.inf))
                minv = jnp.min(d, axis=1, keepdims=True)
                labs.append(jnp.min(
                    jnp.where(d == minv, colg, jnp.int32(1 << 30)),
                    axis=1, keepdims=True) + g * BPG * C)
            return jnp.concatenate(labs, axis=0)

        def moments(lab):
            one = (lab == col).astype(jnp.float32)
            se = lax.dot_general(
                one, selx, (((0,), (0,)), ((), ())),
                preferred_element_type=jnp.float32)
            return se[:, :D], se[:, D:D + 1]

        def cond(st):
            i, _, _, changed = st
            return (i < ITERS) & changed

        def body(st):
            i, centers, lab_prev, _ = st
            lab = labels_of(centers)
            sums, counts = moments(lab)
            newc = jnp.where(counts > 0,
                             sums / jnp.maximum(counts, 1.0), centers)
            return (i + 1, newc, lab,
                    jnp.any(lab != lab_prev))

        st0 = (jnp.int32(0), c0,
               jnp.full((B * K, 1), -1, jnp.int32), jnp.bool_(True))
        _, centers, _, _ = lax.while_loop(cond, body, st0)
        sums, counts = moments(labels_of(centers))
        cf = sums / jnp.maximum(counts, 1.0)
        norm = jnp.sqrt(jnp.sum(cf * cf, axis=1, keepdims=True))
        out_ref[...] = cf / jnp.maximum(norm, jnp.float32(1e-12))


def kernel(patch_token, anomaly_map, prompt_id):
    del prompt_id
    a0 = anomaly_map[:, :, 0].reshape(B, ROWS, LANES)
    a1 = anomaly_map[:, :, 1].reshape(B, ROWS, LANES)
    out = pl.pallas_call(
        _body,
        grid=(2,),
        in_specs=[
            pl.BlockSpec((B, ROWS, LANES), lambda i: (0, 0, 0)),
            pl.BlockSpec((B, ROWS, LANES), lambda i: (0, 0, 0)),
            pl.BlockSpec(memory_space=pl.ANY),
        ],
        out_specs=pl.BlockSpec((B * C, D), lambda i: (0, 0)),
        out_shape=jax.ShapeDtypeStruct((B * C, D), jnp.float32),
        scratch_shapes=[
            pltpu.VMEM((B * K, D), jnp.float32),
            pltpu.VMEM((B, ROWS, LANES), jnp.float32),
            pltpu.VMEM((B, LANES), jnp.int32),
            pltpu.SMEM((B, LANES), jnp.int32),
            pltpu.SemaphoreType.DMA,
        ],
    )(a0, a1, patch_token)
    return out.reshape(B, C, D)


# speedup vs baseline: 1.6029x; 1.1096x over previous
import jax
import jax.numpy as jnp
from jax import lax
from jax.experimental import pallas as pl
from jax.experimental.pallas import tpu as pltpu

B = 8
N = 8192
D = 768
K = 100
C = 20
ITERS = 25
ROWS = 64
LANES = 128


def _body(a0_ref, a1_ref, pt_ref, out_ref, sel_ref, score_ref, idxv_ref,
          idxs_ref, sem):
    pid = pl.program_id(0)

    @pl.when(pid == 0)
    def _topk_gather():
        x0 = a0_ref[...]
        x1 = a1_ref[...]
        m = jnp.maximum(x0, x1)
        e0 = jnp.exp(x0 - m)
        e1 = jnp.exp(x1 - m)
        score_ref[...] = e1 / (e0 + e1)
        lin = (lax.broadcasted_iota(jnp.int32, (B, ROWS, LANES), 1) * LANES
               + lax.broadcasted_iota(jnp.int32, (B, ROWS, LANES), 2))
        lane = lax.broadcasted_iota(jnp.int32, (B, LANES), 1)

        def step(j, acc):
            s = score_ref[...]
            mx = jnp.max(s, axis=(1, 2), keepdims=True)
            cand = jnp.where(s == mx, lin, jnp.int32(1 << 30))
            idx = jnp.min(cand, axis=(1, 2), keepdims=True)
            score_ref[...] = jnp.where(lin == idx, jnp.float32(-1.0), s)
            return jnp.where(lane == j, idx.reshape(B, 1), acc)

        acc = lax.fori_loop(0, K, step,
                            jnp.zeros((B, LANES), jnp.int32), unroll=8)
        idxv_ref[...] = acc
        pltpu.make_async_copy(idxv_ref, idxs_ref, sem).start()
        pltpu.make_async_copy(idxv_ref, idxs_ref, sem).wait()

        for b in range(B):
            def issue(j, _, b=b):
                idx = idxs_ref[b, j]
                pltpu.make_async_copy(
                    pt_ref.at[b, pl.ds(idx, 1), :],
                    sel_ref.at[pl.ds(b * K + j, 1), :],
                    sem,
                ).start()
                return 0

            lax.fori_loop(0, K, issue, 0, unroll=4)

        def drain(j, _):
            pltpu.make_async_copy(
                pt_ref.at[0, pl.ds(0, 1), :],
                sel_ref.at[pl.ds(0, 1), :],
                sem,
            ).wait()
            return 0

        lax.fori_loop(0, B * K, drain, 0, unroll=4)

    @pl.when(pid == 1)
    def _kmeans():
        sel = sel_ref[...]
        ones = jnp.ones((B * K, 1), jnp.float32)
        selx = jnp.concatenate([sel, ones], axis=1)
        col = lax.broadcasted_iota(jnp.int32, (B * K, B * C), 1)
        rowb = lax.broadcasted_iota(jnp.int32, (B * K, B * C), 0) // K
        valid = (col // C) == rowb
        c0 = jnp.concatenate(
            [sel[b * K:b * K + C] for b in range(B)], axis=0)

        G = 4
        BPG = B // G
        colg = lax.broadcasted_iota(jnp.int32, (BPG * K, BPG * C), 1)
        rowbg = lax.broadcasted_iota(jnp.int32, (BPG * K, BPG * C), 0) // K
        validg = (colg // C) == rowbg

        def labels_of(centers):
            c2 = jnp.sum(centers * centers, axis=1, keepdims=True)
            cext = jnp.concatenate([-2.0 * centers, c2], axis=1)
            labs = []
            for g in range(G):
                d = lax.dot_general(
                    selx[g * BPG * K:(g + 1) * BPG * K],
                    cext[g * BPG * C:(g + 1) * BPG * C],
                    (((1,), (1,)), ((), ())),
                    precision=lax.Precision.HIGHEST,
                    preferred_element_type=jnp.float32)
                d = jnp.where(validg, d, jnp.float32(jnp.inf))
                minv = jnp.min(d, axis=1, keepdims=True)
                labs.append(jnp.min(
                    jnp.where(d == minv, colg, jnp.int32(1 << 30)),
                    axis=1, keepdims=True) + g * BPG * C)
            return jnp.concatenate(labs, axis=0)

        def moments(lab):
            one = (lab == col).astype(jnp.float32)
            se = lax.dot_general(
                one, selx, (((0,), (0,)), ((), ())),
                preferred_element_type=jnp.float32)
            return se[:, :D], se[:, D:D + 1]

        def cond(st):
            i, _, _, changed = st
            return (i < ITERS) & changed

        def body(st):
            i, centers, lab_prev, _ = st
            lab = labels_of(centers)
            sums, counts = moments(lab)
            newc = jnp.where(counts > 0,
                             sums / jnp.maximum(counts, 1.0), centers)
            return (i + 1, newc, lab,
                    jnp.any(lab != lab_prev))

        st0 = (jnp.int32(0), c0,
               jnp.full((B * K, 1), -1, jnp.int32), jnp.bool_(True))
        _, centers, _, _ = lax.while_loop(cond, body, st0)
        sums, counts = moments(labels_of(centers))
        cf = sums / jnp.maximum(counts, 1.0)
        norm = jnp.sqrt(jnp.sum(cf * cf, axis=1, keepdims=True))
        out_ref[...] = cf / jnp.maximum(norm, jnp.float32(1e-12))


def kernel(patch_token, anomaly_map, prompt_id):
    del prompt_id
    a0 = anomaly_map[:, :, 0].reshape(B, ROWS, LANES)
    a1 = anomaly_map[:, :, 1].reshape(B, ROWS, LANES)
    out = pl.pallas_call(
        _body,
        grid=(2,),
        in_specs=[
            pl.BlockSpec((B, ROWS, LANES), lambda i: (0, 0, 0)),
            pl.BlockSpec((B, ROWS, LANES), lambda i: (0, 0, 0)),
            pl.BlockSpec(memory_space=pl.ANY),
        ],
        out_specs=pl.BlockSpec((B * C, D), lambda i: (0, 0)),
        out_shape=jax.ShapeDtypeStruct((B * C, D), jnp.float32),
        scratch_shapes=[
            pltpu.VMEM((B * K, D), jnp.float32),
            pltpu.VMEM((B, ROWS, LANES), jnp.float32),
            pltpu.VMEM((B, LANES), jnp.int32),
            pltpu.SMEM((B, LANES), jnp.int32),
            pltpu.SemaphoreType.DMA,
        ],
    )(a0, a1, patch_token)
    return out.reshape(B, C, D)


# speedup vs baseline: 1.6499x; 1.0293x over previous
import jax
import jax.numpy as jnp
from jax import lax
from jax.experimental import pallas as pl
from jax.experimental.pallas import tpu as pltpu

B = 8
N = 8192
D = 768
K = 100
C = 20
ITERS = 25
ROWS = 64
LANES = 128


def _body(a0_ref, a1_ref, pt_ref, out_ref, sel_ref, score_ref, idxv_ref,
          idxs_ref, sem):
    pid = pl.program_id(0)

    @pl.when(pid == 0)
    def _topk_gather():
        x0 = a0_ref[...]
        x1 = a1_ref[...]
        m = jnp.maximum(x0, x1)
        e0 = jnp.exp(x0 - m)
        e1 = jnp.exp(x1 - m)
        score_ref[...] = e1 / (e0 + e1)
        lin = (lax.broadcasted_iota(jnp.int32, (B, ROWS, LANES), 1) * LANES
               + lax.broadcasted_iota(jnp.int32, (B, ROWS, LANES), 2))
        lane = lax.broadcasted_iota(jnp.int32, (B, LANES), 1)

        def step(j, acc):
            s = score_ref[...]
            mx = jnp.max(s, axis=(1, 2), keepdims=True)
            cand = jnp.where(s == mx, lin, jnp.int32(1 << 30))
            idx = jnp.min(cand, axis=(1, 2), keepdims=True)
            score_ref[...] = jnp.where(lin == idx, jnp.float32(-1.0), s)
            return jnp.where(lane == j, idx.reshape(B, 1), acc)

        acc = lax.fori_loop(0, K, step,
                            jnp.zeros((B, LANES), jnp.int32), unroll=20)
        idxv_ref[...] = acc
        pltpu.make_async_copy(idxv_ref, idxs_ref, sem).start()
        pltpu.make_async_copy(idxv_ref, idxs_ref, sem).wait()

        for b in range(B):
            def issue(j, _, b=b):
                idx = idxs_ref[b, j]
                pltpu.make_async_copy(
                    pt_ref.at[b, pl.ds(idx, 1), :],
                    sel_ref.at[pl.ds(b * K + j, 1), :],
                    sem,
                ).start()
                return 0

            lax.fori_loop(0, K, issue, 0, unroll=10)

        def drain(j, _):
            pltpu.make_async_copy(
                pt_ref.at[0, pl.ds(0, 1), :],
                sel_ref.at[pl.ds(0, 1), :],
                sem,
            ).wait()
            return 0

        lax.fori_loop(0, B * K, drain, 0, unroll=10)

    @pl.when(pid == 1)
    def _kmeans():
        sel = sel_ref[...]
        ones = jnp.ones((B * K, 1), jnp.float32)
        selx = jnp.concatenate([sel, ones], axis=1)
        col = lax.broadcasted_iota(jnp.int32, (B * K, B * C), 1)
        rowb = lax.broadcasted_iota(jnp.int32, (B * K, B * C), 0) // K
        valid = (col // C) == rowb
        c0 = jnp.concatenate(
            [sel[b * K:b * K + C] for b in range(B)], axis=0)

        G = 4
        BPG = B // G
        colg = lax.broadcasted_iota(jnp.int32, (BPG * K, BPG * C), 1)
        rowbg = lax.broadcasted_iota(jnp.int32, (BPG * K, BPG * C), 0) // K
        validg = (colg // C) == rowbg

        def labels_of(centers):
            c2 = jnp.sum(centers * centers, axis=1, keepdims=True)
            cext = jnp.concatenate([-2.0 * centers, c2], axis=1)
            labs = []
            for g in range(G):
                d = lax.dot_general(
                    selx[g * BPG * K:(g + 1) * BPG * K],
                    cext[g * BPG * C:(g + 1) * BPG * C],
                    (((1,), (1,)), ((), ())),
                    precision=lax.Precision.HIGHEST,
                    preferred_element_type=jnp.float32)
                d = jnp.where(validg, d, jnp.float32(jnp.inf))
                minv = jnp.min(d, axis=1, keepdims=True)
                labs.append(jnp.min(
                    jnp.where(d == minv, colg, jnp.int32(1 << 30)),
                    axis=1, keepdims=True) + g * BPG * C)
            return jnp.concatenate(labs, axis=0)

        def moments(lab):
            one = (lab == col).astype(jnp.float32)
            se = lax.dot_general(
                one, selx, (((0,), (0,)), ((), ())),
                preferred_element_type=jnp.float32)
            return se[:, :D], se[:, D:D + 1]

        def cond(st):
            i, _, _, changed = st
            return (i < ITERS) & changed

        def body(st):
            i, centers, lab_prev, _ = st
            lab = labels_of(centers)
            sums, counts = moments(lab)
            newc = jnp.where(counts > 0,
                             sums / jnp.maximum(counts, 1.0), centers)
            return (i + 1, newc, lab,
                    jnp.any(lab != lab_prev))

        st0 = (jnp.int32(0), c0,
               jnp.full((B * K, 1), -1, jnp.int32), jnp.bool_(True))
        _, centers, _, _ = lax.while_loop(cond, body, st0)
        sums, counts = moments(labels_of(centers))
        cf = sums / jnp.maximum(counts, 1.0)
        norm = jnp.sqrt(jnp.sum(cf * cf, axis=1, keepdims=True))
        out_ref[...] = cf / jnp.maximum(norm, jnp.float32(1e-12))


def kernel(patch_token, anomaly_map, prompt_id):
    del prompt_id
    a0 = anomaly_map[:, :, 0].reshape(B, ROWS, LANES)
    a1 = anomaly_map[:, :, 1].reshape(B, ROWS, LANES)
    out = pl.pallas_call(
        _body,
        grid=(2,),
        in_specs=[
            pl.BlockSpec((B, ROWS, LANES), lambda i: (0, 0, 0)),
            pl.BlockSpec((B, ROWS, LANES), lambda i: (0, 0, 0)),
            pl.BlockSpec(memory_space=pl.ANY),
        ],
        out_specs=pl.BlockSpec((B * C, D), lambda i: (0, 0)),
        out_shape=jax.ShapeDtypeStruct((B * C, D), jnp.float32),
        scratch_shapes=[
            pltpu.VMEM((B * K, D), jnp.float32),
            pltpu.VMEM((B, ROWS, LANES), jnp.float32),
            pltpu.VMEM((B, LANES), jnp.int32),
            pltpu.SMEM((B, LANES), jnp.int32),
            pltpu.SemaphoreType.DMA,
        ],
    )(a0, a1, patch_token)
    return out.reshape(B, C, D)
